# Initial kernel scaffold; baseline (speedup 1.0000x reference)
import jax
import jax.numpy as jnp
from jax.experimental import pallas as pl


def kernel(x, edge_index_0, edge_weight_0, edge_index_1, edge_weight_1, W0, W1, b):
    raise NotImplementedError("write your pallas kernel here")


# speedup vs baseline: 17.3533x; 17.3533x over previous
import functools

import jax
import jax.numpy as jnp
from jax import lax
from jax.experimental import pallas as pl
from jax.experimental.pallas import tpu as pltpu
from jax.experimental.pallas import tpu_sc as plsc

B, N, D = 2, 10000, 256
DH = 128
P = 2
E2 = 320000
NS = 16
NC = 2
EPS = E2 // NS
CH = 80
NCH = EPS // CH
SEGS = 5
SCH = NCH // SEGS
HALF = N // NC
RPS = 320
HPAD = NS * RPS
OC = 8
NOC = RPS // OC
TROWS = P * B * 2 * N


_RB = 1000


def _mm_body(x_ref, w_ref, o_ref):
    xb = x_ref[0]
    h0 = jnp.dot(xb, w_ref[0], preferred_element_type=jnp.float32)
    h1 = jnp.dot(xb, w_ref[1], preferred_element_type=jnp.float32)
    o_ref[0, 0, 0] = h0[:, :DH]
    o_ref[0, 0, 1] = h1[:, :DH]
    o_ref[1, 0, 0] = h0[:, DH:]
    o_ref[1, 0, 1] = h1[:, DH:]


_matmul = pl.pallas_call(
    _mm_body,
    grid=(B, N // _RB),
    in_specs=[
        pl.BlockSpec((1, _RB, D), lambda b, i: (b, i, 0)),
        pl.BlockSpec((2, D, D), lambda b, i: (0, 0, 0)),
    ],
    out_specs=pl.BlockSpec((P, 1, 2, _RB, DH), lambda b, i: (0, b, 0, i, 0)),
    out_shape=jax.ShapeDtypeStruct((P, B, 2, N, DH), jnp.float32),
)


_mesh = plsc.VectorSubcoreMesh(core_axis_name="c", subcore_axis_name="s")


@functools.partial(
    pl.kernel,
    mesh=_mesh,
    out_type=jax.ShapeDtypeStruct((B, N, D), jnp.float32),
    scratch_types=[
        pltpu.VMEM((SCH, CH), jnp.int32),
        pltpu.VMEM((SCH, CH), jnp.int32),
        pltpu.VMEM((SCH, CH), jnp.float32),
        pltpu.VMEM((CH, DH), jnp.float32),
        pltpu.VMEM((OC, DH), jnp.float32),
        pltpu.VMEM((OC, DH), jnp.float32),
        pltpu.VMEM((D,), jnp.float32),
        pltpu.VMEM_SHARED((HPAD, DH), jnp.float32),
        pltpu.SemaphoreType.DMA,
    ],
)
def _sc_aggregate(pre_hbm, src_hbm, dst_hbm, w_hbm, bias_hbm, out_hbm,
                  src_buf, idx_buf, wm_buf, rows,
                  oc_buf, zbuf, bias_buf, acc, sem):
    c = lax.axis_index("c")
    s = lax.axis_index("s")
    sc_off = c * HALF

    pltpu.sync_copy(bias_hbm, bias_buf)

    zv = jnp.zeros((16,), jnp.float32)
    for r in range(OC):
        for j in range(DH // 16):
            zbuf[r, pl.ds(j * 16, 16)] = zv

    for b in range(B):
        for p in range(P):
            def _zero_step(k, carry):
                pltpu.sync_copy(zbuf, acc.at[pl.ds(s * RPS + k * OC, OC), :])
                return carry

            lax.fori_loop(0, NOC, _zero_step, 0)
            plsc.subcore_barrier()

            def _seg_step(seg, carry0):
                pltpu.sync_copy(src_hbm.at[b * P + p, s, seg], src_buf)
                pltpu.sync_copy(dst_hbm.at[s, seg], idx_buf)
                pltpu.sync_copy(w_hbm.at[s, seg], wm_buf)

                def _chunk_step(ci, carry):
                    for g in range(CH // 16):
                        sl = pl.ds(g * 16, 16)
                        d = idx_buf[ci, sl] - sc_off
                        inr = (d >= 0) & (d < HALF)
                        idx_buf[ci, sl] = jnp.where(inr, d, 0)
                        wm_buf[ci, sl] = jnp.where(inr, wm_buf[ci, sl], 0.0)

                    pltpu.async_copy(pre_hbm.at[src_buf.at[ci]], rows, sem).wait()

                    def _wmul(g, c2):
                        wv = wm_buf[ci, pl.ds(g * 16, 16)]
                        for l in range(16):
                            w = wv[l]
                            e = g * 16 + l
                            for j in range(DH // 16):
                                sl = pl.ds(j * 16, 16)
                                rows[e, sl] = rows[e, sl] * w
                        return c2

                    lax.fori_loop(0, CH // 16, _wmul, 0)
                    pltpu.sync_copy(rows, acc.at[idx_buf.at[ci]], add=True)
                    return carry

                lax.fori_loop(0, SCH, _chunk_step, 0)
                return carry0

            lax.fori_loop(0, SEGS, _seg_step, 0)
            plsc.subcore_barrier()

            def _read_step(k, carry):
                l0 = s * RPS + k * OC

                @pl.when(l0 < HALF)
                def _():
                    pltpu.sync_copy(acc.at[pl.ds(l0, OC), :], oc_buf)
                    for r in range(OC):
                        for j in range(DH // 16):
                            sl = pl.ds(j * 16, 16)
                            v = oc_buf[r, sl] + bias_buf[pl.ds(p * DH + j * 16, 16)]
                            oc_buf[r, sl] = jnp.maximum(v, 0.0)
                    pltpu.sync_copy(
                        oc_buf,
                        out_hbm.at[b, pl.ds(sc_off + l0, OC), pl.ds(p * DH, DH)])

                return carry

            lax.fori_loop(0, NOC, _read_step, 0)
            plsc.subcore_barrier()


def kernel(x, edge_index_0, edge_weight_0, edge_index_1, edge_weight_1, W0, W1, b):
    Ws = jnp.stack([W0, W1])
    pre = _matmul(x, Ws)
    pre_flat = pre.reshape(TROWS, DH)

    src_all = jnp.concatenate([edge_index_0[1], edge_index_1[1] + N])
    offs = (jnp.arange(B)[:, None] * 2 * N + jnp.arange(P)[None, :] * 2 * B * N)
    src4 = (src_all[None, None, :] + offs[:, :, None])
    src4 = src4.reshape(B * P, NS, SEGS, SCH, CH)
    dst_all = jnp.concatenate([edge_index_0[0], edge_index_1[0]])
    dst_all = dst_all.reshape(NS, SEGS, SCH, CH)
    w_all = jnp.concatenate([edge_weight_0, edge_weight_1])
    w_all = w_all.reshape(NS, SEGS, SCH, CH)

    return _sc_aggregate(pre_flat, src4, dst_all, w_all, b)


# speedup vs baseline: 20.9614x; 1.2079x over previous
import functools

import jax
import jax.numpy as jnp
from jax import lax
from jax.experimental import pallas as pl
from jax.experimental.pallas import tpu as pltpu
from jax.experimental.pallas import tpu_sc as plsc

B, N, D = 2, 10000, 256
DH = 128
P = 2
E2 = 320000
NS = 16
NC = 2
EPS = E2 // NS
CH = 80
NCH = EPS // CH
SEGS = 5
SCH = NCH // SEGS
HALF = N // NC
RPS = 320
HPAD = NS * RPS
OC = 8
NOC = RPS // OC
TROWS = P * B * 2 * N


_RB = 1000


def _mm_body(x_ref, w_ref, o_ref):
    xb = x_ref[0]
    h0 = jnp.dot(xb, w_ref[0], preferred_element_type=jnp.float32)
    h1 = jnp.dot(xb, w_ref[1], preferred_element_type=jnp.float32)
    o_ref[0, 0, 0] = h0[:, :DH]
    o_ref[0, 0, 1] = h1[:, :DH]
    o_ref[1, 0, 0] = h0[:, DH:]
    o_ref[1, 0, 1] = h1[:, DH:]


_matmul = pl.pallas_call(
    _mm_body,
    grid=(B, N // _RB),
    in_specs=[
        pl.BlockSpec((1, _RB, D), lambda b, i: (b, i, 0)),
        pl.BlockSpec((2, D, D), lambda b, i: (0, 0, 0)),
    ],
    out_specs=pl.BlockSpec((P, 1, 2, _RB, DH), lambda b, i: (0, b, 0, i, 0)),
    out_shape=jax.ShapeDtypeStruct((P, B, 2, N, DH), jnp.float32),
)


_mesh = plsc.VectorSubcoreMesh(core_axis_name="c", subcore_axis_name="s")


@functools.partial(
    pl.kernel,
    mesh=_mesh,
    out_type=jax.ShapeDtypeStruct((B, N, D), jnp.float32),
    scratch_types=[
        pltpu.VMEM((SCH, CH), jnp.int32),
        pltpu.VMEM((SCH, CH), jnp.int32),
        pltpu.VMEM((SCH, CH), jnp.float32),
        pltpu.VMEM((CH, DH), jnp.float32),
        pltpu.VMEM((CH, DH), jnp.float32),
        pltpu.VMEM((OC, DH), jnp.float32),
        pltpu.VMEM((OC, DH), jnp.float32),
        pltpu.VMEM((D,), jnp.float32),
        pltpu.VMEM_SHARED((HPAD, DH), jnp.float32),
        pltpu.SemaphoreType.DMA,
        pltpu.SemaphoreType.DMA,
    ],
)
def _sc_aggregate(pre_hbm, src_hbm, dst_hbm, w_hbm, bias_hbm, out_hbm,
                  src_buf, idx_buf, wm_buf, rows0, rows1,
                  oc_buf, zbuf, bias_buf, acc, sem0, sem1):
    c = lax.axis_index("c")
    s = lax.axis_index("s")
    sc_off = c * HALF

    pltpu.sync_copy(bias_hbm, bias_buf)

    zv = jnp.zeros((16,), jnp.float32)
    for r in range(OC):
        for j in range(DH // 16):
            zbuf[r, pl.ds(j * 16, 16)] = zv

    for b in range(B):
        for p in range(P):
            def _zero_step(k, carry):
                pltpu.sync_copy(zbuf, acc.at[pl.ds(s * RPS + k * OC, OC), :])
                return carry

            lax.fori_loop(0, NOC, _zero_step, 0)
            plsc.subcore_barrier()

            def _seg_step(seg, carry0):
                pltpu.sync_copy(src_hbm.at[b * P + p, s, seg], src_buf)
                pltpu.sync_copy(dst_hbm.at[s, seg], idx_buf)
                pltpu.sync_copy(w_hbm.at[s, seg], wm_buf)

                def _mask_chunk(cc):
                    for g in range(CH // 16):
                        sl = pl.ds(g * 16, 16)
                        d = idx_buf[cc, sl] - sc_off
                        inr = (d >= 0) & (d < HALF)
                        idx_buf[cc, sl] = jnp.where(inr, d, 0)
                        wm_buf[cc, sl] = jnp.where(inr, wm_buf[cc, sl], 0.0)

                def _wmul(rbuf, cc):
                    def _wmul_g(g, c2):
                        wv = wm_buf[cc, pl.ds(g * 16, 16)]
                        for l in range(16):
                            w = wv[l]
                            e = g * 16 + l
                            for j in range(DH // 16):
                                sl = pl.ds(j * 16, 16)
                                rbuf[e, sl] = rbuf[e, sl] * w
                        return c2

                    lax.fori_loop(0, CH // 16, _wmul_g, 0)

                def _chunk_step(ci2, carry):
                    c0 = ci2 * 2
                    c1 = c0 + 1
                    _mask_chunk(c0)
                    _mask_chunk(c1)
                    g0 = pltpu.async_copy(pre_hbm.at[src_buf.at[c0]], rows0, sem0)
                    g1 = pltpu.async_copy(pre_hbm.at[src_buf.at[c1]], rows1, sem1)
                    g0.wait()
                    _wmul(rows0, c0)
                    pltpu.sync_copy(rows0, acc.at[idx_buf.at[c0]], add=True)
                    g1.wait()
                    _wmul(rows1, c1)
                    pltpu.sync_copy(rows1, acc.at[idx_buf.at[c1]], add=True)
                    return carry

                lax.fori_loop(0, SCH // 2, _chunk_step, 0)
                return carry0

            lax.fori_loop(0, SEGS, _seg_step, 0)
            plsc.subcore_barrier()

            def _read_step(k, carry):
                l0 = s * RPS + k * OC

                @pl.when(l0 < HALF)
                def _():
                    pltpu.sync_copy(acc.at[pl.ds(l0, OC), :], oc_buf)
                    for r in range(OC):
                        for j in range(DH // 16):
                            sl = pl.ds(j * 16, 16)
                            v = oc_buf[r, sl] + bias_buf[pl.ds(p * DH + j * 16, 16)]
                            oc_buf[r, sl] = jnp.maximum(v, 0.0)
                    pltpu.sync_copy(
                        oc_buf,
                        out_hbm.at[b, pl.ds(sc_off + l0, OC), pl.ds(p * DH, DH)])

                return carry

            lax.fori_loop(0, NOC, _read_step, 0)
            plsc.subcore_barrier()


def kernel(x, edge_index_0, edge_weight_0, edge_index_1, edge_weight_1, W0, W1, b):
    Ws = jnp.stack([W0, W1])
    pre = _matmul(x, Ws)
    pre_flat = pre.reshape(TROWS, DH)

    src_all = jnp.concatenate([edge_index_0[1], edge_index_1[1] + N])
    offs = (jnp.arange(B)[:, None] * 2 * N + jnp.arange(P)[None, :] * 2 * B * N)
    src4 = (src_all[None, None, :] + offs[:, :, None])
    src4 = src4.reshape(B * P, NS, SEGS, SCH, CH)
    dst_all = jnp.concatenate([edge_index_0[0], edge_index_1[0]])
    dst_all = dst_all.reshape(NS, SEGS, SCH, CH)
    w_all = jnp.concatenate([edge_weight_0, edge_weight_1])
    w_all = w_all.reshape(NS, SEGS, SCH, CH)

    return _sc_aggregate(pre_flat, src4, dst_all, w_all, b)
